# baseline (device time: 43442 ns/iter reference)
import jax
import jax.numpy as jnp
from jax import lax
from jax.experimental import pallas as pl
from jax.experimental.pallas import tpu as pltpu


def _swap_with_z_peer(x, dest2d):

    def body(x_ref, d_ref, px_ref, pd_ref, send_sems, recv_sems):
        mx = lax.axis_index("x")
        my = lax.axis_index("y")
        mz = lax.axis_index("z")
        peer = (mx, my, 1 - mz)

        barrier_sem = pltpu.get_barrier_semaphore()
        pl.semaphore_signal(
            barrier_sem, inc=1, device_id=peer,
            device_id_type=pl.DeviceIdType.MESH,
        )
        pl.semaphore_wait(barrier_sem, 1)

        rdma_x = pltpu.make_async_remote_copy(
            src_ref=x_ref,
            dst_ref=px_ref,
            send_sem=send_sems.at[0],
            recv_sem=recv_sems.at[0],
            device_id=peer,
            device_id_type=pl.DeviceIdType.MESH,
        )
        rdma_d = pltpu.make_async_remote_copy(
            src_ref=d_ref,
            dst_ref=pd_ref,
            send_sem=send_sems.at[1],
            recv_sem=recv_sems.at[1],
            device_id=peer,
            device_id_type=pl.DeviceIdType.MESH,
        )
        rdma_x.start()
        rdma_d.start()
        rdma_x.wait()
        rdma_d.wait()

    return pl.pallas_call(
        body,
        out_shape=(
            jax.ShapeDtypeStruct(x.shape, x.dtype),
            jax.ShapeDtypeStruct(dest2d.shape, dest2d.dtype),
        ),
        in_specs=[
            pl.BlockSpec(memory_space=pltpu.VMEM),
            pl.BlockSpec(memory_space=pltpu.VMEM),
        ],
        out_specs=(
            pl.BlockSpec(memory_space=pltpu.VMEM),
            pl.BlockSpec(memory_space=pltpu.VMEM),
        ),
        scratch_shapes=[
            pltpu.SemaphoreType.DMA((2,)),
            pltpu.SemaphoreType.DMA((2,)),
        ],
        compiler_params=pltpu.CompilerParams(collective_id=0),
    )(x, dest2d)


def kernel(x, dest):
    m, n = x.shape
    mz = lax.axis_index("z")

    dest2d = dest.reshape(8, -1)
    peer_x, peer_dest2d = _swap_with_z_peer(x, dest2d)
    peer_dest = peer_dest2d.reshape(-1)

    full_x = jnp.zeros((2 * m, n), x.dtype)
    full_x = lax.dynamic_update_slice(full_x, x, (mz * m, 0))
    full_x = lax.dynamic_update_slice(full_x, peer_x, ((1 - mz) * m, 0))
    full_dest = jnp.zeros((2 * m,), dest.dtype)
    full_dest = lax.dynamic_update_slice(full_dest, dest, (mz * m,))
    full_dest = lax.dynamic_update_slice(full_dest, peer_dest, ((1 - mz) * m,))

    order = jnp.argsort(full_dest, stable=True)
    my_idx = lax.dynamic_slice(order, (mz * m,), (m,))
    return full_x[my_idx]


# device time: 31202 ns/iter; 1.3923x vs baseline; 1.3923x over previous
import jax
import jax.numpy as jnp
from jax import lax
from jax.experimental import pallas as pl
from jax.experimental.pallas import tpu as pltpu

_BITS = list(range(10, 2, -1))


def kernel(x, dest):
    m, n = x.shape
    mz = lax.axis_index("z")

    a = jnp.sum((dest != mz).astype(jnp.int32))
    e = (-a) % 8
    k = m - a

    order = jnp.argsort(dest, stable=True)
    sx = jnp.take(x, order, axis=0)

    i = jnp.arange(m, dtype=jnp.int32)
    sb_idx = jnp.where(mz == 0, jnp.clip(k + i, 0, m - 1),
                       jnp.clip(i - e, 0, m - 1))
    sb = jnp.take(sx, sb_idx, axis=0)

    a_arr = a.reshape(1)

    def body(a_ref, sx_ref, sb_ref, out_ref, send_sems, recv_sems):
        mx = lax.axis_index("x")
        my = lax.axis_index("y")
        mz = lax.axis_index("z")
        peer = (mx, my, 1 - mz)
        a = a_ref[0]
        e = (-a) % 8
        A = a + e
        dst0 = jnp.where(mz == 0, 0, m - A)
        rbase = jnp.where(mz == 0, m - A, 0)

        out_ref[...] = sx_ref[...]

        barrier_sem = pltpu.get_barrier_semaphore()
        pl.semaphore_signal(
            barrier_sem, inc=1, device_id=peer,
            device_id_type=pl.DeviceIdType.MESH,
        )
        pl.semaphore_wait(barrier_sem, 1)

        descs = []
        off = jnp.int32(0)
        for idx, b in enumerate(_BITS):
            s = 1 << b
            bit = (A >> b) & 1

            def mk_send(off=off, s=s, idx=idx):
                return pltpu.make_async_remote_copy(
                    src_ref=sb_ref.at[pl.ds(pl.multiple_of(off, 8), s), :],
                    dst_ref=out_ref.at[
                        pl.ds(pl.multiple_of(dst0 + off, 8), s), :],
                    send_sem=send_sems.at[idx],
                    recv_sem=recv_sems.at[idx],
                    device_id=peer,
                    device_id_type=pl.DeviceIdType.MESH,
                )

            def mk_recv(off=off, s=s, idx=idx):
                return pltpu.make_async_remote_copy(
                    src_ref=sb_ref.at[pl.ds(pl.multiple_of(off, 8), s), :],
                    dst_ref=out_ref.at[
                        pl.ds(pl.multiple_of(rbase + off, 8), s), :],
                    send_sem=send_sems.at[idx],
                    recv_sem=recv_sems.at[idx],
                    device_id=peer,
                    device_id_type=pl.DeviceIdType.MESH,
                )

            @pl.when(bit == 1)
            def _(mk=mk_send):
                mk().start()

            descs.append((bit, mk_send, mk_recv))
            off = off + bit * s

        for bit, mk_send, mk_recv in descs:
            @pl.when(bit == 1)
            def _(mk_send=mk_send, mk_recv=mk_recv):
                mk_send().wait_send()
                mk_recv().wait_recv()

        @pl.when(e > 0)
        def _():
            bb = pl.multiple_of(jnp.where(mz == 0, m - A, A - 8), 8)
            band = out_ref[pl.ds(bb, 8), :]
            sxband = sx_ref[pl.ds(bb, 8), :]
            j = lax.broadcasted_iota(jnp.int32, (8, n), 0)
            jj = 7 * mz + (1 - 2 * mz) * j
            out_ref[pl.ds(bb, 8), :] = jnp.where(jj < e, sxband, band)

    return pl.pallas_call(
        body,
        out_shape=jax.ShapeDtypeStruct((m, n), x.dtype),
        in_specs=[
            pl.BlockSpec(memory_space=pltpu.SMEM),
            pl.BlockSpec(memory_space=pltpu.VMEM),
            pl.BlockSpec(memory_space=pltpu.VMEM),
        ],
        out_specs=pl.BlockSpec(memory_space=pltpu.VMEM),
        scratch_shapes=[
            pltpu.SemaphoreType.DMA((len(_BITS),)),
            pltpu.SemaphoreType.DMA((len(_BITS),)),
        ],
        compiler_params=pltpu.CompilerParams(collective_id=0),
    )(a_arr, sx, sb)


# device time: 28924 ns/iter; 1.5019x vs baseline; 1.0788x over previous
import jax
import jax.numpy as jnp
from jax import lax
from jax.experimental import pallas as pl
from jax.experimental.pallas import tpu as pltpu

_BITS = list(range(10, 2, -1))


def kernel(x, dest):
    m, n = x.shape
    mz = lax.axis_index("z")

    a = jnp.sum((dest != mz).astype(jnp.int32))
    order = jnp.argsort(dest, stable=True)
    sx = jnp.take(x, order, axis=0)
    a_arr = a.reshape(1)

    def body(a_ref, sx_ref, out_ref, recv_ref, send_sems, recv_sems):
        mx = lax.axis_index("x")
        my = lax.axis_index("y")
        mz = lax.axis_index("z")
        peer = (mx, my, 1 - mz)
        a = a_ref[0]
        e = (-a) % 8
        A = a + e
        k = m - a
        sbase = jnp.where(mz == 0, m - A, 0)

        barrier_sem = pltpu.get_barrier_semaphore()
        pl.semaphore_signal(
            barrier_sem, inc=1, device_id=peer,
            device_id_type=pl.DeviceIdType.MESH,
        )
        pl.semaphore_wait(barrier_sem, 1)

        descs = []
        off = jnp.int32(0)
        for idx, b in enumerate(_BITS):
            s = 1 << b
            bit = (A >> b) & 1

            def mk(off=off, s=s, idx=idx):
                return pltpu.make_async_remote_copy(
                    src_ref=sx_ref.at[
                        pl.ds(pl.multiple_of(sbase + off, 8), s), :],
                    dst_ref=recv_ref.at[
                        pl.ds(pl.multiple_of(off, 8), s), :],
                    send_sem=send_sems.at[idx],
                    recv_sem=recv_sems.at[idx],
                    device_id=peer,
                    device_id_type=pl.DeviceIdType.MESH,
                )

            @pl.when(bit == 1)
            def _(mk=mk):
                mk().start()

            descs.append((bit, mk))
            off = off + bit * s

        for bit, mk in descs:
            @pl.when(bit == 1)
            def _(mk=mk):
                d = mk()
                d.wait_send()
                d.wait_recv()

        mzi = mz.astype(jnp.int32)
        shift = jnp.where(mz == 0, k % m, (m - e) % m)
        rolled = pltpu.roll(recv_ref[...], shift, 0)
        p = lax.broadcasted_iota(jnp.int32, (m, n), 0)
        q = (1 - mzi) * ((m - 1) - p) + mzi * p
        out_ref[...] = jnp.where(q < a, rolled, sx_ref[...])

    return pl.pallas_call(
        body,
        out_shape=jax.ShapeDtypeStruct((m, n), x.dtype),
        in_specs=[
            pl.BlockSpec(memory_space=pltpu.SMEM),
            pl.BlockSpec(memory_space=pltpu.VMEM),
        ],
        out_specs=pl.BlockSpec(memory_space=pltpu.VMEM),
        scratch_shapes=[
            pltpu.VMEM((m, n), x.dtype),
            pltpu.SemaphoreType.DMA((len(_BITS),)),
            pltpu.SemaphoreType.DMA((len(_BITS),)),
        ],
        compiler_params=pltpu.CompilerParams(collective_id=0),
    )(a_arr, sx)


# device time: 26703 ns/iter; 1.6269x vs baseline; 1.0832x over previous
import jax
import jax.numpy as jnp
from jax import lax
from jax.experimental import pallas as pl
from jax.experimental.pallas import tpu as pltpu

_BITS = list(range(10, 2, -1))


def kernel(x, dest):
    m, n = x.shape
    mz = lax.axis_index("z")

    i = jnp.arange(m, dtype=jnp.int32)
    c = jnp.cumsum(dest)
    n1 = c[m - 1]
    n0 = m - n1
    zeros_incl = (i + 1) - c
    pos = jnp.where(dest == 0, zeros_incl - 1, n0 + c - 1)
    sx = jnp.zeros_like(x).at[pos].set(x, mode="drop", unique_indices=True)
    a = jnp.where(mz == 0, n1, n0)
    a_arr = a.reshape(1).astype(jnp.int32)

    def body(a_ref, sx_ref, out_ref, recv_ref, send_sems, recv_sems):
        mx = lax.axis_index("x")
        my = lax.axis_index("y")
        mz = lax.axis_index("z")
        peer = (mx, my, 1 - mz)
        a = a_ref[0]
        e = (-a) % 8
        A = a + e
        k = m - a
        sbase = jnp.where(mz == 0, m - A, 0)

        barrier_sem = pltpu.get_barrier_semaphore()
        pl.semaphore_signal(
            barrier_sem, inc=1, device_id=peer,
            device_id_type=pl.DeviceIdType.MESH,
        )
        pl.semaphore_wait(barrier_sem, 1)

        descs = []
        off = jnp.int32(0)
        for idx, b in enumerate(_BITS):
            s = 1 << b
            bit = (A >> b) & 1

            def mk(off=off, s=s, idx=idx):
                return pltpu.make_async_remote_copy(
                    src_ref=sx_ref.at[
                        pl.ds(pl.multiple_of(sbase + off, 8), s), :],
                    dst_ref=recv_ref.at[
                        pl.ds(pl.multiple_of(off, 8), s), :],
                    send_sem=send_sems.at[idx],
                    recv_sem=recv_sems.at[idx],
                    device_id=peer,
                    device_id_type=pl.DeviceIdType.MESH,
                )

            @pl.when(bit == 1)
            def _(mk=mk):
                mk().start()

            descs.append((bit, mk))
            off = off + bit * s

        for bit, mk in descs:
            @pl.when(bit == 1)
            def _(mk=mk):
                d = mk()
                d.wait_send()
                d.wait_recv()

        mzi = mz.astype(jnp.int32)
        shift = jnp.where(mz == 0, k % m, (m - e) % m)
        rolled = pltpu.roll(recv_ref[...], shift, 0)
        p = lax.broadcasted_iota(jnp.int32, (m, n), 0)
        q = (1 - mzi) * ((m - 1) - p) + mzi * p
        out_ref[...] = jnp.where(q < a, rolled, sx_ref[...])

    return pl.pallas_call(
        body,
        out_shape=jax.ShapeDtypeStruct((m, n), x.dtype),
        in_specs=[
            pl.BlockSpec(memory_space=pltpu.SMEM),
            pl.BlockSpec(memory_space=pltpu.VMEM),
        ],
        out_specs=pl.BlockSpec(memory_space=pltpu.VMEM),
        scratch_shapes=[
            pltpu.VMEM((m, n), x.dtype),
            pltpu.SemaphoreType.DMA((len(_BITS),)),
            pltpu.SemaphoreType.DMA((len(_BITS),)),
        ],
        compiler_params=pltpu.CompilerParams(collective_id=0),
    )(a_arr, sx)


# device time: 25922 ns/iter; 1.6759x vs baseline; 1.0301x over previous
import jax
import jax.numpy as jnp
from jax import lax
from jax.experimental import pallas as pl
from jax.experimental.pallas import tpu as pltpu

_BITS = list(range(10, 2, -1))


def kernel(x, dest):
    m, n = x.shape
    mz = lax.axis_index("z")

    i = jnp.arange(m, dtype=jnp.int32)
    c = jnp.cumsum(dest)
    n1 = c[m - 1]
    n0 = m - n1
    z = (i + 1) - c
    a = jnp.where(mz == 0, n1, n0)
    e = (-a) % 8

    keep_pos = jnp.where(mz == 0, z - 1, n0 + c - 1)
    send_rank = jnp.where(mz == 0, c - 1, z - 1)
    send_off = jnp.where(mz == 0, 0, e)
    pos = jnp.where(dest == mz, keep_pos, m + send_off + send_rank)
    sall = jnp.zeros((2 * m, n), x.dtype).at[pos].set(
        x, mode="drop", unique_indices=True)
    a_arr = a.reshape(1).astype(jnp.int32)

    def body(a_ref, sall_ref, out_ref, send_sems, recv_sems, copy_sems):
        mx = lax.axis_index("x")
        my = lax.axis_index("y")
        mz = lax.axis_index("z")
        peer = (mx, my, 1 - mz)
        a = a_ref[0]
        e = (-a) % 8
        A = a + e
        rbase = jnp.where(mz == 0, m - A, 0)
        dst0 = jnp.where(mz == 0, 0, m - A)
        lbase = jnp.where(mz == 0, 0, A)

        barrier_sem = pltpu.get_barrier_semaphore()
        pl.semaphore_signal(
            barrier_sem, inc=1, device_id=peer,
            device_id_type=pl.DeviceIdType.MESH,
        )

        copies = []
        loff = jnp.int32(0)
        klen = m - A
        for idx, b in enumerate(_BITS):
            s = 1 << b
            bit = (klen >> b) & 1

            def mkc(loff=loff, s=s, idx=idx):
                return pltpu.make_async_copy(
                    sall_ref.at[pl.ds(pl.multiple_of(lbase + loff, 8), s), :],
                    out_ref.at[pl.ds(pl.multiple_of(lbase + loff, 8), s), :],
                    copy_sems.at[idx],
                )

            @pl.when(bit == 1)
            def _(mkc=mkc):
                mkc().start()

            copies.append((bit, mkc))
            loff = loff + bit * s

        pl.semaphore_wait(barrier_sem, 1)

        descs = []
        off = jnp.int32(0)
        for idx, b in enumerate(_BITS):
            s = 1 << b
            bit = (A >> b) & 1

            def mk(off=off, s=s, idx=idx):
                return pltpu.make_async_remote_copy(
                    src_ref=sall_ref.at[pl.ds(pl.multiple_of(m + off, 8), s), :],
                    dst_ref=out_ref.at[
                        pl.ds(pl.multiple_of(dst0 + off, 8), s), :],
                    send_sem=send_sems.at[idx],
                    recv_sem=recv_sems.at[idx],
                    device_id=peer,
                    device_id_type=pl.DeviceIdType.MESH,
                )

            def mkr(off=off, s=s, idx=idx):
                return pltpu.make_async_remote_copy(
                    src_ref=sall_ref.at[pl.ds(pl.multiple_of(m + off, 8), s), :],
                    dst_ref=out_ref.at[
                        pl.ds(pl.multiple_of(rbase + off, 8), s), :],
                    send_sem=send_sems.at[idx],
                    recv_sem=recv_sems.at[idx],
                    device_id=peer,
                    device_id_type=pl.DeviceIdType.MESH,
                )

            @pl.when(bit == 1)
            def _(mk=mk):
                mk().start()

            descs.append((bit, mk, mkr))
            off = off + bit * s

        for bit, mkc in copies:
            @pl.when(bit == 1)
            def _(mkc=mkc):
                mkc().wait()

        for bit, mk, mkr in descs:
            @pl.when(bit == 1)
            def _(mk=mk, mkr=mkr):
                mk().wait_send()
                mkr().wait_recv()

        @pl.when(e > 0)
        def _():
            bb = pl.multiple_of(jnp.where(mz == 0, m - A, A - 8), 8)
            band = out_ref[pl.ds(bb, 8), :]
            keep = sall_ref[pl.ds(bb, 8), :]
            j = lax.broadcasted_iota(jnp.int32, (8, n), 0)
            jj = 7 * mz + (1 - 2 * mz) * j
            out_ref[pl.ds(bb, 8), :] = jnp.where(jj < e, keep, band)

    return pl.pallas_call(
        body,
        out_shape=jax.ShapeDtypeStruct((m, n), x.dtype),
        in_specs=[
            pl.BlockSpec(memory_space=pltpu.SMEM),
            pl.BlockSpec(memory_space=pltpu.VMEM),
        ],
        out_specs=pl.BlockSpec(memory_space=pltpu.VMEM),
        scratch_shapes=[
            pltpu.SemaphoreType.DMA((len(_BITS),)),
            pltpu.SemaphoreType.DMA((len(_BITS),)),
            pltpu.SemaphoreType.DMA((len(_BITS),)),
        ],
        compiler_params=pltpu.CompilerParams(collective_id=0),
    )(a_arr, sall)
